# baseline (speedup 1.0000x reference)
import jax
import jax.numpy as jnp
from jax.experimental import pallas as pl
from jax.experimental.pallas import tpu as pltpu

_MT = 200
_DEPTH = 5


def _hi_dot(x, w):
    return jax.lax.dot_general(
        x, w, (((1,), (0,)), ((), ())),
        precision=jax.lax.Precision.HIGHEST,
        preferred_element_type=jnp.float32)


def _body(x_ref, w1_ref, b1_ref, w2_ref, b2_ref, adj_ref, o_ref,
          buf_ref, sem_ref, s1_ref, h2_ref):
    n = x_ref.shape[0]
    nb = n // _MT

    def copy(slot, idx):
        return pltpu.make_async_copy(
            adj_ref.at[pl.ds((idx % nb) * _MT, _MT), :],
            buf_ref.at[slot],
            sem_ref.at[slot],
        )

    for s in range(_DEPTH):
        copy(s, s).start()

    def s1_chunk(k, _):
        x = x_ref[pl.ds(k * 2000, 2000), :]
        s1_ref[pl.ds(k * 2000, 2000), :] = (
            _hi_dot(x, w1_ref[...]).astype(jnp.bfloat16))
        return 0

    jax.lax.fori_loop(0, n // 2000, s1_chunk, 0)

    def fetch_and_mm(b, rhs_ref):
        slot = jax.lax.rem(b, _DEPTH)
        copy(slot, b).wait()
        a = buf_ref[slot].astype(jnp.bfloat16)
        acc = jnp.dot(a, rhs_ref[...], preferred_element_type=jnp.float32)
        nxt = b + _DEPTH

        @pl.when(nxt < 2 * nb)
        def _():
            copy(slot, nxt).start()

        return acc

    def loop1(p, _):
        acc0 = fetch_and_mm(2 * p, s1_ref)
        acc1 = fetch_and_mm(2 * p + 1, s1_ref)
        acc = jnp.concatenate([acc0, acc1], axis=0)
        h = jnp.maximum(acc + b1_ref[...], 0.0)
        h2_ref[pl.ds(p * 2 * _MT, 2 * _MT), :] = (
            _hi_dot(h, w2_ref[...]).astype(jnp.bfloat16))
        return 0

    jax.lax.fori_loop(0, nb // 2, loop1, 0)

    def loop2(b, _):
        x = fetch_and_mm(b, h2_ref) + b2_ref[...]
        m = jnp.max(x, axis=1, keepdims=True)
        s = x - m
        o_ref[pl.ds(jax.lax.rem(b, nb) * _MT, _MT), :] = (
            s - jnp.log(jnp.sum(jnp.exp(s), axis=1, keepdims=True)))
        return 0

    jax.lax.fori_loop(nb, 2 * nb, loop2, 0)


def kernel(feature, adj, W1, b1, W2, b2):
    n, d_in = feature.shape
    d_hid = W1.shape[1]
    d_out = W2.shape[1]

    return pl.pallas_call(
        _body,
        in_specs=[
            pl.BlockSpec(memory_space=pltpu.MemorySpace.VMEM),
            pl.BlockSpec(memory_space=pltpu.MemorySpace.VMEM),
            pl.BlockSpec(memory_space=pltpu.MemorySpace.VMEM),
            pl.BlockSpec(memory_space=pltpu.MemorySpace.VMEM),
            pl.BlockSpec(memory_space=pltpu.MemorySpace.VMEM),
            pl.BlockSpec(memory_space=pltpu.MemorySpace.HBM),
        ],
        out_specs=pl.BlockSpec(memory_space=pltpu.MemorySpace.VMEM),
        out_shape=jax.ShapeDtypeStruct((n, d_out), jnp.float32),
        scratch_shapes=[
            pltpu.VMEM((_DEPTH, _MT, n), jnp.float32),
            pltpu.SemaphoreType.DMA((_DEPTH,)),
            pltpu.VMEM((n, d_hid), jnp.bfloat16),
            pltpu.VMEM((n, d_out), jnp.bfloat16),
        ],
    )(feature, W1, b1.reshape(1, -1), W2, b2.reshape(1, -1), adj)


# speedup vs baseline: 1.0074x; 1.0074x over previous
import jax
import jax.numpy as jnp
from jax.experimental import pallas as pl
from jax.experimental.pallas import tpu as pltpu

_MT = 200
_DEPTH = 4


def _hi_dot(x, w):
    return jax.lax.dot_general(
        x, w, (((1,), (0,)), ((), ())),
        precision=jax.lax.Precision.HIGHEST,
        preferred_element_type=jnp.float32)


def _body(x_ref, w1_ref, b1_ref, w2_ref, b2_ref, adj_ref, o_ref,
          buf_ref, sem_ref, s1_ref, h2_ref):
    n = x_ref.shape[0]
    nb = n // _MT

    def copy(slot, idx):
        return pltpu.make_async_copy(
            adj_ref.at[pl.ds((idx % nb) * _MT, _MT), :],
            buf_ref.at[slot],
            sem_ref.at[slot],
        )

    for s in range(_DEPTH):
        copy(s, s).start()

    def s1_chunk(k, _):
        x = x_ref[pl.ds(k * 2000, 2000), :]
        s1_ref[pl.ds(k * 2000, 2000), :] = (
            _hi_dot(x, w1_ref[...]).astype(jnp.bfloat16))
        return 0

    jax.lax.fori_loop(0, n // 2000, s1_chunk, 0)

    def fetch_and_mm(b, rhs_ref):
        slot = jax.lax.rem(b, _DEPTH)
        copy(slot, b).wait()
        a = buf_ref[slot].astype(jnp.bfloat16)
        nxt = b + _DEPTH

        @pl.when(nxt < 2 * nb)
        def _():
            copy(slot, nxt).start()

        return jnp.dot(a, rhs_ref[...], preferred_element_type=jnp.float32)

    def loop1(p, _):
        acc0 = fetch_and_mm(2 * p, s1_ref)
        acc1 = fetch_and_mm(2 * p + 1, s1_ref)
        acc = jnp.concatenate([acc0, acc1], axis=0)
        h = jnp.maximum(acc + b1_ref[...], 0.0)
        h2_ref[pl.ds(p * 2 * _MT, 2 * _MT), :] = (
            _hi_dot(h, w2_ref[...]).astype(jnp.bfloat16))
        return 0

    jax.lax.fori_loop(0, nb // 2, loop1, 0)

    def loop2(p, _):
        x0 = fetch_and_mm(nb + 2 * p, h2_ref)
        x1 = fetch_and_mm(nb + 2 * p + 1, h2_ref)
        x = jnp.concatenate([x0, x1], axis=0) + b2_ref[...]
        m = jnp.max(x, axis=1, keepdims=True)
        s = x - m
        o_ref[pl.ds(p * 2 * _MT, 2 * _MT), :] = (
            s - jnp.log(jnp.sum(jnp.exp(s), axis=1, keepdims=True)))
        return 0

    jax.lax.fori_loop(0, nb // 2, loop2, 0)


def kernel(feature, adj, W1, b1, W2, b2):
    n, d_in = feature.shape
    d_hid = W1.shape[1]
    d_out = W2.shape[1]

    return pl.pallas_call(
        _body,
        in_specs=[
            pl.BlockSpec(memory_space=pltpu.MemorySpace.VMEM),
            pl.BlockSpec(memory_space=pltpu.MemorySpace.VMEM),
            pl.BlockSpec(memory_space=pltpu.MemorySpace.VMEM),
            pl.BlockSpec(memory_space=pltpu.MemorySpace.VMEM),
            pl.BlockSpec(memory_space=pltpu.MemorySpace.VMEM),
            pl.BlockSpec(memory_space=pltpu.MemorySpace.HBM),
        ],
        out_specs=pl.BlockSpec(memory_space=pltpu.MemorySpace.VMEM),
        out_shape=jax.ShapeDtypeStruct((n, d_out), jnp.float32),
        scratch_shapes=[
            pltpu.VMEM((_DEPTH, _MT, n), jnp.float32),
            pltpu.SemaphoreType.DMA((_DEPTH,)),
            pltpu.VMEM((n, d_hid), jnp.bfloat16),
            pltpu.VMEM((n, d_out), jnp.bfloat16),
        ],
    )(feature, W1, b1.reshape(1, -1), W2, b2.reshape(1, -1), adj)


# speedup vs baseline: 1.0082x; 1.0008x over previous
import jax
import jax.numpy as jnp
from jax.experimental import pallas as pl
from jax.experimental.pallas import tpu as pltpu

_MT = 200
_DEPTH = 4


def _hi_dot(x, w):
    return jax.lax.dot_general(
        x, w, (((1,), (0,)), ((), ())),
        precision=jax.lax.Precision.HIGHEST,
        preferred_element_type=jnp.float32)


def _body(x_ref, w1_ref, b1_ref, w2_ref, b2_ref, adj_ref, o_ref,
          buf_ref, sem_ref, s1_ref, h2_ref):
    n = x_ref.shape[0]
    nb = n // _MT

    def copy(slot, idx):
        return pltpu.make_async_copy(
            adj_ref.at[pl.ds((idx % nb) * _MT, _MT), :],
            buf_ref.at[slot],
            sem_ref.at[slot],
        )

    for s in range(_DEPTH):
        copy(s, s).start()

    def s1_chunk(k, _):
        x = x_ref[pl.ds(k * 2000, 2000), :]
        s1_ref[pl.ds(k * 2000, 2000), :] = (
            _hi_dot(x, w1_ref[...]).astype(jnp.bfloat16))
        return 0

    jax.lax.fori_loop(0, n // 2000, s1_chunk, 0)

    def fetch_and_mm(b, rhs_ref):
        slot = jax.lax.rem(b, _DEPTH)
        copy(slot, b).wait()
        a = buf_ref[slot].astype(jnp.bfloat16)
        acc = jnp.dot(a, rhs_ref[...], preferred_element_type=jnp.float32)
        nxt = b + _DEPTH

        @pl.when(nxt < 2 * nb)
        def _():
            copy(slot, nxt).start()

        return acc

    def loop1(p, _):
        acc0 = fetch_and_mm(2 * p, s1_ref)
        acc1 = fetch_and_mm(2 * p + 1, s1_ref)
        acc = jnp.concatenate([acc0, acc1], axis=0)
        h = jnp.maximum(acc + b1_ref[...], 0.0)
        h2_ref[pl.ds(p * 2 * _MT, 2 * _MT), :] = (
            _hi_dot(h, w2_ref[...]).astype(jnp.bfloat16))
        return 0

    jax.lax.fori_loop(0, nb // 2, loop1, 0)

    def loop2(b, _):
        x = fetch_and_mm(b, h2_ref) + b2_ref[...]
        m = jnp.max(x, axis=1, keepdims=True)
        s = x - m
        o_ref[pl.ds(jax.lax.rem(b, nb) * _MT, _MT), :] = (
            s - jnp.log(jnp.sum(jnp.exp(s), axis=1, keepdims=True)))
        return 0

    jax.lax.fori_loop(nb, 2 * nb, loop2, 0)


def kernel(feature, adj, W1, b1, W2, b2):
    n, d_in = feature.shape
    d_hid = W1.shape[1]
    d_out = W2.shape[1]

    return pl.pallas_call(
        _body,
        in_specs=[
            pl.BlockSpec(memory_space=pltpu.MemorySpace.VMEM),
            pl.BlockSpec(memory_space=pltpu.MemorySpace.VMEM),
            pl.BlockSpec(memory_space=pltpu.MemorySpace.VMEM),
            pl.BlockSpec(memory_space=pltpu.MemorySpace.VMEM),
            pl.BlockSpec(memory_space=pltpu.MemorySpace.VMEM),
            pl.BlockSpec(memory_space=pltpu.MemorySpace.HBM),
        ],
        out_specs=pl.BlockSpec(memory_space=pltpu.MemorySpace.VMEM),
        out_shape=jax.ShapeDtypeStruct((n, d_out), jnp.float32),
        scratch_shapes=[
            pltpu.VMEM((_DEPTH, _MT, n), jnp.float32),
            pltpu.SemaphoreType.DMA((_DEPTH,)),
            pltpu.VMEM((n, d_hid), jnp.bfloat16),
            pltpu.VMEM((n, d_out), jnp.bfloat16),
        ],
    )(feature, W1, b1.reshape(1, -1), W2, b2.reshape(1, -1), adj)
